# baseline (speedup 1.0000x reference)
import functools

import jax
import jax.numpy as jnp
from jax import lax
from jax.experimental import pallas as pl
from jax.experimental.pallas import tpu as pltpu
from jax.experimental.pallas import tpu_sc as plsc

NUM_POSITIONS = 1024
HIDDEN = 768
HALF = HIDDEN // 2
BATCH = 64
SEQ = 1024

NC = 2
NS = 16
NW = NC * NS

TOTAL = BATCH * SEQ
BPW = TOTAL // NW
CHUNK = 32
NCHUNK = BPW // CHUNK
LANES = 16
QROW = HALF // LANES

_mesh = plsc.VectorSubcoreMesh(core_axis_name="c", subcore_axis_name="s")


@functools.partial(
    pl.kernel,
    mesh=_mesh,
    out_type=jax.ShapeDtypeStruct((TOTAL, HIDDEN), jnp.int32),
    scratch_types=[
        pltpu.VMEM((NCHUNK, CHUNK), jnp.int32),
        pltpu.VMEM((2, CHUNK, HALF), jnp.int32),
        pltpu.VMEM((2, CHUNK, HIDDEN), jnp.int32),
    ] + [pltpu.SemaphoreType.DMA] * 4,
)
def _gather_rows(ids_hbm, ptab_hbm, out_hbm, idx_v, packed_v, stage_v,
                 g0, g1, w0, w1):
    gs = (g0, g1)
    ws = (w0, w1)
    wid = lax.axis_index("s") * NC + lax.axis_index("c")
    base = wid * BPW
    pltpu.sync_copy(ids_hbm.at[wid], idx_v)

    def start_gather(ci, b):
        pltpu.async_copy(ptab_hbm.at[idx_v.at[ci]], packed_v.at[b], gs[b])

    def wait_gather(b):
        pltpu.make_async_copy(
            ptab_hbm.at[idx_v.at[0]], packed_v.at[b], gs[b]).wait()

    def start_write(ci, b):
        pltpu.async_copy(
            stage_v.at[b], out_hbm.at[pl.ds(base + ci * CHUNK, CHUNK)], ws[b])

    def wait_write(b):
        pltpu.make_async_copy(
            stage_v.at[b], out_hbm.at[pl.ds(base, CHUNK)], ws[b]).wait()

    def expand(b):
        src = packed_v.at[b]
        dst = stage_v.at[b]

        @plsc.parallel_loop(0, CHUNK, unroll=8)
        def _(r):
            for q in range(QROW):
                x = src[r, pl.ds(LANES * q, LANES)]
                dst[r, pl.ds(2 * LANES * q, LANES)] = lax.shift_left(x, 16)
                dst[r, pl.ds(2 * LANES * q + LANES, LANES)] = (
                    x & jnp.int32(-65536))

    start_gather(0, 0)

    def step(o, carry):
        for r in range(2):
            ci = 2 * o + r
            b = r

            @pl.when(ci + 1 < NCHUNK)
            def _(ci=ci, b=b):
                start_gather(ci + 1, 1 - b)

            @pl.when(ci >= 2)
            def _(b=b):
                wait_write(b)

            wait_gather(b)
            expand(b)
            start_write(ci, b)
        return carry

    lax.fori_loop(0, NCHUNK // 2, step, 0)
    wait_write(0)
    wait_write(1)


def kernel(position_ids, table):
    ids = jnp.reshape(position_ids.astype(jnp.int32), (NW, NCHUNK, CHUNK))
    t3 = jnp.reshape(table.astype(jnp.bfloat16),
                     (NUM_POSITIONS, HALF // LANES, 2, LANES))
    lo = lax.bitcast_convert_type(t3[:, :, 0, :], jnp.uint16).astype(jnp.uint32)
    hi = lax.bitcast_convert_type(t3[:, :, 1, :], jnp.uint16).astype(jnp.uint32)
    packed = lax.bitcast_convert_type(lo | (hi << 16), jnp.int32)
    packed = jnp.reshape(packed, (NUM_POSITIONS, HALF))
    out = _gather_rows(ids, packed)
    return lax.bitcast_convert_type(
        jnp.reshape(out, (BATCH, SEQ, HIDDEN)), jnp.float32)


# speedup vs baseline: 1.5785x; 1.5785x over previous
import functools

import jax
import jax.numpy as jnp
from jax import lax
from jax.experimental import pallas as pl
from jax.experimental.pallas import tpu as pltpu
from jax.experimental.pallas import tpu_sc as plsc

NUM_POSITIONS = 1024
HIDDEN = 768
BATCH = 64
SEQ = 1024

NC = 2
NS = 16
NW = NC * NS

TOTAL = BATCH * SEQ
BPW = TOTAL // NW
CHUNK = 64
NCHUNK = BPW // CHUNK

_mesh = plsc.VectorSubcoreMesh(core_axis_name="c", subcore_axis_name="s")


@functools.partial(
    pl.kernel,
    mesh=_mesh,
    out_type=jax.ShapeDtypeStruct((TOTAL, HIDDEN), jnp.float32),
    scratch_types=[
        pltpu.VMEM((NCHUNK, CHUNK), jnp.int32),
        pltpu.VMEM((2, CHUNK, HIDDEN), jnp.float32),
        pltpu.SemaphoreType.DMA,
        pltpu.SemaphoreType.DMA,
        pltpu.SemaphoreType.DMA,
        pltpu.SemaphoreType.DMA,
    ],
)
def _gather_rows(ids_hbm, table_hbm, out_hbm, idx_v, rows_v, g0, g1, w0, w1):
    wid = lax.axis_index("s") * NC + lax.axis_index("c")
    base = wid * BPW
    gs = (g0, g1)
    ws = (w0, w1)
    pltpu.sync_copy(ids_hbm.at[wid], idx_v)

    def start_gather(ci, b):
        pltpu.async_copy(table_hbm.at[idx_v.at[ci]], rows_v.at[b], gs[b])

    def wait_gather(b):
        pltpu.make_async_copy(
            table_hbm.at[idx_v.at[0]], rows_v.at[b], gs[b]).wait()

    def start_write(ci, b):
        pltpu.async_copy(
            rows_v.at[b], out_hbm.at[pl.ds(base + ci * CHUNK, CHUNK)], ws[b])

    def wait_write(b):
        pltpu.make_async_copy(
            rows_v.at[b], out_hbm.at[pl.ds(base, CHUNK)], ws[b]).wait()

    def step(o, carry):
        later = o > 0

        @pl.when(later)
        def _():
            wait_write(0)

        start_gather(2 * o, 0)

        @pl.when(later)
        def _():
            wait_gather(1)
            start_write(2 * o - 1, 1)

        @pl.when(later)
        def _():
            wait_write(1)

        start_gather(2 * o + 1, 1)
        wait_gather(0)
        start_write(2 * o, 0)
        return carry

    lax.fori_loop(0, NCHUNK // 2, step, 0)
    wait_gather(1)
    start_write(NCHUNK - 1, 1)
    wait_write(0)
    wait_write(1)


def kernel(position_ids, table):
    ids = jnp.reshape(position_ids.astype(jnp.int32), (NW, NCHUNK, CHUNK))
    out = _gather_rows(ids, table)
    return jnp.reshape(out, (BATCH, SEQ, HIDDEN))
